# baseline (speedup 1.0000x reference)
import functools

import jax
import jax.numpy as jnp
from jax import lax
from jax.experimental import pallas as pl
from jax.experimental.pallas import tpu as pltpu

G, N = 4, 128
L, H, NH = 4, 32, 16
NB = 6
EDIM, NTE = 4, 2
V_SP, V_ACT, V_EDG, V_NT = 512, 128, 64, 128

TP = 1024
NTILE = G * N * N // TP
IB = TP // N
NIB = N // IB

NV = 4 * NB + 5
_C_BLNG = 0
_C_BLNB = NB
_C_B1 = 2 * NB
_C_B2 = 3 * NB
_C_NORMG = 4 * NB
_C_NORMB = 4 * NB + 1
_C_FC1B = 4 * NB + 2
_C_RESW = 4 * NB + 3
_C_RESB = 4 * NB + 4


def _gelu(x):
    return 0.5 * x * (1.0 + lax.erf(x * 0.7071067811865476))


def _ln_t(x, cs, be, gcol=None, bcol=None):
    f32 = jnp.float32
    mu4 = jnp.dot(cs, x, preferred_element_type=f32)
    m24 = jnp.dot(cs, x * x, preferred_element_type=f32)
    rs4 = lax.rsqrt(m24 - mu4 * mu4 + 1e-5)
    mub = jnp.dot(be, mu4 * rs4, preferred_element_type=f32)
    rsb = jnp.dot(be, rs4, preferred_element_type=f32)
    xn = x * rsb - mub
    if gcol is None:
        return xn
    return xn * gcol + bcol


def _body(idx_ref, tsp_ref, tac_ref, ted_ref, tnt_ref, mats_ref, vecs_ref,
          fc2t_ref, fc2b_ref, cs_ref, be_ref, o_ref):
    rows = idx_ref[0]
    sp_row = rows[0:1]
    f32 = jnp.float32

    def onehot(row, v):
        io = lax.broadcasted_iota(jnp.int32, (v, TP), 0)
        return (io == row).astype(f32)

    acc = jnp.dot(tsp_ref[...], onehot(sp_row, V_SP),
                  preferred_element_type=f32)
    acc += jnp.dot(tac_ref[...], onehot(rows[1:2], V_ACT),
                   preferred_element_type=f32)
    ed_cnt = (onehot(rows[2:3], V_EDG) + onehot(rows[3:4], V_EDG)
              + onehot(rows[4:5], V_EDG) + onehot(rows[5:6], V_EDG))
    acc += 0.25 * jnp.dot(ted_ref[...], ed_cnt, preferred_element_type=f32)
    nt_cnt = onehot(rows[6:7], V_NT) + onehot(rows[7:8], V_NT)
    acc += 0.5 * jnp.dot(tnt_ref[...], nt_cnt, preferred_element_type=f32)
    res_row = lax.bitcast_convert_type(rows[8:9], f32)
    acc += vecs_ref[:, _C_RESW:_C_RESW + 1] * res_row \
        + vecs_ref[:, _C_RESB:_C_RESB + 1]
    x = acc * 0.2

    for i in range(NB):
        h = _ln_t(x, cs_ref[...], be_ref[...])
        h = jnp.dot(mats_ref[i], h, preferred_element_type=f32) \
            + vecs_ref[:, _C_B1 + i:_C_B1 + i + 1]
        h = _gelu(h)
        h = jnp.dot(mats_ref[NB + i], h, preferred_element_type=f32) \
            + vecs_ref[:, _C_B2 + i:_C_B2 + i + 1]
        x = x + h

    x = _ln_t(x, cs_ref[...], be_ref[...],
              vecs_ref[:, _C_NORMG:_C_NORMG + 1],
              vecs_ref[:, _C_NORMB:_C_NORMB + 1])
    x = _gelu(x)
    x = jnp.dot(mats_ref[2 * NB], x, preferred_element_type=f32) \
        + vecs_ref[:, _C_FC1B:_C_FC1B + 1]
    x = _gelu(x)
    y = jnp.dot(fc2t_ref[...], x, preferred_element_type=f32) + fc2b_ref[...]

    mask = sp_row > 0
    y = jnp.where(mask, y, 0.0)
    o_ref[...] = y.reshape(L, NH, IB, N)[:, None]


@jax.jit
def kernel(spatial_pos, edge_long, action_pos, res_pos, node_type_edge,
           spatial_tab, action_tab, edge_tab, ntype_tab, res_w, res_b,
           bln_g, bln_b, bfc1_w, bfc1_b, bfc2_w, bfc2_b,
           norm_g, norm_b, fc1_w, fc1_b, fc2_w, fc2_b, t):
    f32 = jnp.float32

    flat = lambda a: a.reshape(-1)
    idx_rows = jnp.stack([
        flat(spatial_pos), flat(action_pos),
        flat(edge_long[..., 0]), flat(edge_long[..., 1]),
        flat(edge_long[..., 2]), flat(edge_long[..., 3]),
        flat(node_type_edge[..., 0]), flat(node_type_edge[..., 1]),
        lax.bitcast_convert_type(flat(res_pos), jnp.int32),
    ])
    idx_pack = idx_rows.reshape(9, NTILE, TP).transpose(1, 0, 2)

    tsp = spatial_tab.at[0].set(0.0).T
    tac = action_tab.at[0].set(0.0).T
    ted = edge_tab.at[0].set(0.0).T
    tnt = ntype_tab.at[0].set(0.0).T

    eye = jnp.eye(L, dtype=f32)
    bd = jax.vmap(lambda w: jnp.kron(eye, w.T))
    w1g = bln_g[:, :, None] * bfc1_w
    mats = jnp.concatenate([bd(w1g), bd(bfc2_w),
                            jnp.kron(eye, fc1_w.T)[None]],
                           axis=0)
    fc2t = jnp.kron(eye, fc2_w.T)
    fc2b = jnp.tile(fc2_b, L)[:, None]

    tile4 = lambda v: jnp.tile(v, L)
    vec_cols = ([tile4(bln_g[i]) for i in range(NB)]
                + [tile4(bln_b[i]) for i in range(NB)]
                + [tile4(bfc1_b[i] + bln_b[i] @ bfc1_w[i]) for i in range(NB)]
                + [tile4(bfc2_b[i]) for i in range(NB)]
                + [tile4(norm_g), tile4(norm_b), tile4(fc1_b),
                   res_w.reshape(-1), res_b])
    vecs = jnp.stack(vec_cols, axis=1)

    chunk_of = jnp.arange(L * H, dtype=jnp.int32) // H
    cs = (chunk_of[None, :] == jnp.arange(L)[:, None]).astype(f32) / H
    be = (chunk_of[:, None] == jnp.arange(L)[None, :]).astype(f32)

    grid = (G, NIB)
    z = pl.pallas_call(
        _body,
        grid=grid,
        in_specs=[
            pl.BlockSpec((1, 9, TP), lambda g, ib: (g * NIB + ib, 0, 0)),
            pl.BlockSpec((L * H, V_SP), lambda g, ib: (0, 0)),
            pl.BlockSpec((L * H, V_ACT), lambda g, ib: (0, 0)),
            pl.BlockSpec((L * H, V_EDG), lambda g, ib: (0, 0)),
            pl.BlockSpec((L * H, V_NT), lambda g, ib: (0, 0)),
            pl.BlockSpec((2 * NB + 1, L * H, L * H), lambda g, ib: (0, 0, 0)),
            pl.BlockSpec((L * H, NV), lambda g, ib: (0, 0)),
            pl.BlockSpec((L * NH, L * H), lambda g, ib: (0, 0)),
            pl.BlockSpec((L * NH, 1), lambda g, ib: (0, 0)),
            pl.BlockSpec((L, L * H), lambda g, ib: (0, 0)),
            pl.BlockSpec((L * H, L), lambda g, ib: (0, 0)),
        ],
        out_specs=pl.BlockSpec((L, 1, NH, IB, N),
                               lambda g, ib: (0, g, 0, ib, 0)),
        out_shape=jax.ShapeDtypeStruct((L, G, NH, N, N), f32),
    )(idx_pack, tsp, tac, ted, tnt, mats, vecs, fc2t, fc2b, cs, be)

    out = jnp.zeros((L, G, NH, N + 1, N + 1), dtype=f32)
    out = out.at[:, :, :, 1:, 1:].set(z)
    out = out.at[:, :, :, 0, 0].set(jnp.broadcast_to(t[0][:, None, :], (L, G, NH)))
    out = out.at[:, :, :, 0, 1:].set(
        jnp.broadcast_to(t[1][:, None, :, None], (L, G, NH, N)))
    out = out.at[:, :, :, 1:, 0].set(
        jnp.broadcast_to(t[2][:, None, :, None], (L, G, NH, N)))
    return out


# speedup vs baseline: 1.2555x; 1.2555x over previous
import functools

import jax
import jax.numpy as jnp
from jax import lax
from jax.experimental import pallas as pl
from jax.experimental.pallas import tpu as pltpu

G, N = 4, 128
L, H, NH = 4, 32, 16
NB = 6
EDIM, NTE = 4, 2
V_SP, V_ACT, V_EDG, V_NT = 512, 128, 64, 128

TP = 1024
NHALF = 2
TT = TP * NHALF
NTILE = G * N * N // TT
IB = TP // N
NIB = N // (IB * NHALF)

NV = 4 * NB + 5
_C_BLNG = 0
_C_BLNB = NB
_C_B1 = 2 * NB
_C_B2 = 3 * NB
_C_NORMG = 4 * NB
_C_NORMB = 4 * NB + 1
_C_FC1B = 4 * NB + 2
_C_RESW = 4 * NB + 3
_C_RESB = 4 * NB + 4


def _gelu(x):
    return 0.5 * x * (1.0 + lax.erf(x * 0.7071067811865476))


def _ln_t(x, gcol=None, bcol=None):
    x3 = x.reshape(L, H, TP)
    mu = jnp.mean(x3, axis=1, keepdims=True)
    xc = x3 - mu
    var = jnp.mean(xc * xc, axis=1, keepdims=True)
    xn = (xc * lax.rsqrt(var + 1e-5)).reshape(L * H, TP)
    if gcol is None:
        return xn
    return xn * gcol + bcol


def _half(rows, tsp, tac, ted, tnt, mats, vecs, fc2t, fc2b):
    f32 = jnp.float32
    sp_row = rows[0:1]

    def onehot(row, v):
        io = lax.broadcasted_iota(jnp.int32, (v, TP), 0)
        return (io == row).astype(f32)

    acc = jnp.dot(tsp, onehot(sp_row, V_SP), preferred_element_type=f32)
    acc += jnp.dot(tac, onehot(rows[1:2], V_ACT), preferred_element_type=f32)
    ed_cnt = (onehot(rows[2:3], V_EDG) + onehot(rows[3:4], V_EDG)
              + onehot(rows[4:5], V_EDG) + onehot(rows[5:6], V_EDG))
    acc += 0.25 * jnp.dot(ted, ed_cnt, preferred_element_type=f32)
    nt_cnt = onehot(rows[6:7], V_NT) + onehot(rows[7:8], V_NT)
    acc += 0.5 * jnp.dot(tnt, nt_cnt, preferred_element_type=f32)
    res_row = lax.bitcast_convert_type(rows[8:9], f32)
    acc += vecs[:, _C_RESW:_C_RESW + 1] * res_row \
        + vecs[:, _C_RESB:_C_RESB + 1]
    x = acc * 0.2

    for i in range(NB):
        h = _ln_t(x)
        h = jnp.dot(mats[i], h, preferred_element_type=f32) \
            + vecs[:, _C_B1 + i:_C_B1 + i + 1]
        h = _gelu(h)
        h = jnp.dot(mats[NB + i], h, preferred_element_type=f32) \
            + vecs[:, _C_B2 + i:_C_B2 + i + 1]
        x = x + h

    x = _ln_t(x, vecs[:, _C_NORMG:_C_NORMG + 1],
              vecs[:, _C_NORMB:_C_NORMB + 1])
    x = _gelu(x)
    x = jnp.dot(mats[2 * NB], x, preferred_element_type=f32) \
        + vecs[:, _C_FC1B:_C_FC1B + 1]
    x = _gelu(x)
    y = jnp.dot(fc2t, x, preferred_element_type=f32) + fc2b
    y = jnp.where(sp_row > 0, y, 0.0)
    return y.reshape(L, NH, IB, N)


def _body(idx_ref, tsp_ref, tac_ref, ted_ref, tnt_ref, mats_ref, vecs_ref,
          fc2t_ref, fc2b_ref, o_ref):
    args = (tsp_ref[...], tac_ref[...], ted_ref[...], tnt_ref[...],
            mats_ref, vecs_ref[...], fc2t_ref[...], fc2b_ref[...])
    ys = [_half(idx_ref[0, :, h * TP:(h + 1) * TP], *args)
          for h in range(NHALF)]
    o_ref[...] = jnp.concatenate(ys, axis=2)[:, None]


@jax.jit
def kernel(spatial_pos, edge_long, action_pos, res_pos, node_type_edge,
           spatial_tab, action_tab, edge_tab, ntype_tab, res_w, res_b,
           bln_g, bln_b, bfc1_w, bfc1_b, bfc2_w, bfc2_b,
           norm_g, norm_b, fc1_w, fc1_b, fc2_w, fc2_b, t):
    f32 = jnp.float32

    flat = lambda a: a.reshape(-1)
    idx_rows = jnp.stack([
        flat(spatial_pos), flat(action_pos),
        flat(edge_long[..., 0]), flat(edge_long[..., 1]),
        flat(edge_long[..., 2]), flat(edge_long[..., 3]),
        flat(node_type_edge[..., 0]), flat(node_type_edge[..., 1]),
        lax.bitcast_convert_type(flat(res_pos), jnp.int32),
    ])
    idx_pack = idx_rows.reshape(9, NTILE, TT).transpose(1, 0, 2)

    tsp = spatial_tab.at[0].set(0.0).T
    tac = action_tab.at[0].set(0.0).T
    ted = edge_tab.at[0].set(0.0).T
    tnt = ntype_tab.at[0].set(0.0).T

    eye = jnp.eye(L, dtype=f32)
    bd = jax.vmap(lambda w: jnp.kron(eye, w.T))
    w1g = bln_g[:, :, None] * bfc1_w
    mats = jnp.concatenate([bd(w1g), bd(bfc2_w),
                            jnp.kron(eye, fc1_w.T)[None]],
                           axis=0)
    fc2t = jnp.kron(eye, fc2_w.T)
    fc2b = jnp.tile(fc2_b, L)[:, None]

    tile4 = lambda v: jnp.tile(v, L)
    vec_cols = ([tile4(bln_g[i]) for i in range(NB)]
                + [tile4(bln_b[i]) for i in range(NB)]
                + [tile4(bfc1_b[i] + bln_b[i] @ bfc1_w[i]) for i in range(NB)]
                + [tile4(bfc2_b[i]) for i in range(NB)]
                + [tile4(norm_g), tile4(norm_b), tile4(fc1_b),
                   res_w.reshape(-1), res_b])
    vecs = jnp.stack(vec_cols, axis=1)

    grid = (G, NIB)
    z = pl.pallas_call(
        _body,
        grid=grid,
        in_specs=[
            pl.BlockSpec((1, 9, TT), lambda g, ib: (g * NIB + ib, 0, 0)),
            pl.BlockSpec((L * H, V_SP), lambda g, ib: (0, 0)),
            pl.BlockSpec((L * H, V_ACT), lambda g, ib: (0, 0)),
            pl.BlockSpec((L * H, V_EDG), lambda g, ib: (0, 0)),
            pl.BlockSpec((L * H, V_NT), lambda g, ib: (0, 0)),
            pl.BlockSpec((2 * NB + 1, L * H, L * H), lambda g, ib: (0, 0, 0)),
            pl.BlockSpec((L * H, NV), lambda g, ib: (0, 0)),
            pl.BlockSpec((L * NH, L * H), lambda g, ib: (0, 0)),
            pl.BlockSpec((L * NH, 1), lambda g, ib: (0, 0)),
        ],
        out_specs=pl.BlockSpec((L, 1, NH, IB * NHALF, N),
                               lambda g, ib: (0, g, 0, ib, 0)),
        out_shape=jax.ShapeDtypeStruct((L, G, NH, N, N), f32),
    )(idx_pack, tsp, tac, ted, tnt, mats, vecs, fc2t, fc2b)

    out = jnp.zeros((L, G, NH, N + 1, N + 1), dtype=f32)
    out = out.at[:, :, :, 1:, 1:].set(z)
    out = out.at[:, :, :, 0, 0].set(jnp.broadcast_to(t[0][:, None, :], (L, G, NH)))
    out = out.at[:, :, :, 0, 1:].set(
        jnp.broadcast_to(t[1][:, None, :, None], (L, G, NH, N)))
    out = out.at[:, :, :, 1:, 0].set(
        jnp.broadcast_to(t[2][:, None, :, None], (L, G, NH, N)))
    return out
